# baseline (speedup 1.0000x reference)
import functools

import jax
import jax.numpy as jnp
from jax import lax
from jax.experimental import pallas as pl
from jax.experimental.pallas import tpu as pltpu
from jax.experimental.pallas import tpu_sc as plsc

N = 50000
D = 256
G = 128

NCORES = 2
NTILES = 16
DC = D // NCORES
LANES = 16
NV = DC // LANES
ROWS_PER_TILE = 3128
OVERLAP = NTILES * ROWS_PER_TILE - N
CHUNK = 128
NFULL = ROWS_PER_TILE // CHUNK
TAIL = ROWS_PER_TILE - NFULL * CHUNK
NCHUNK = NFULL + 1
CPG = 2
STAGE = CPG * CHUNK
NGATHER = NFULL // CPG


def _body(
    feats_hbm, ids_hbm, out_hbm,
    ids_v, fbuf, zbuf, runbuf, acc, sem0, sem1, semf,
):
    cid = lax.axis_index("c")
    sid = lax.axis_index("s")
    col0 = cid * DC
    base = jnp.minimum(sid * ROWS_PER_TILE, N - ROWS_PER_TILE)
    sems = (sem0, sem1)

    def gather(g, b):
        return pltpu.make_async_copy(
            feats_hbm.at[pl.ds(base + g * STAGE, STAGE), pl.ds(col0, DC)],
            fbuf.at[b],
            sems[b],
        )

    gather(0, 0).start()
    gather(1, 1).start()

    zero = jnp.zeros((LANES,), jnp.float32)
    for r in range(8):
        for j in range(NV):
            zbuf[r, pl.ds(j * LANES, LANES)] = zero
    pltpu.sync_copy(zbuf, acc.at[pl.ds(sid * 8, 8)])
    for r in range(2 * LANES):
        for j in range(NV):
            runbuf[r, pl.ds(j * LANES, LANES)] = zero

    pltpu.sync_copy(ids_hbm.at[sid], ids_v)
    plsc.subcore_barrier()

    trash_idx = jnp.full((LANES,), G, jnp.int32)
    pltpu.async_copy(
        runbuf.at[pl.ds(0, LANES)], acc.at[trash_idx], semf, add=True
    )

    def drain():
        pltpu.make_async_copy(
            runbuf.at[pl.ds(0, LANES)], acc.at[trash_idx], semf
        ).wait()

    def scatter_flush(prev, sel):
        drain()
        pltpu.async_copy(
            runbuf.at[pl.ds(sel * LANES, LANES)],
            acc.at[jnp.full((LANES,), prev, jnp.int32)],
            semf,
            add=True,
        )

    def absorb(rid, loadfn, carry):
        prev, sel = carry
        change = rid != prev
        newrun = jnp.logical_and(change, prev >= 0)
        nsel = jnp.where(change, 1 - sel, sel)

        @pl.when(newrun)
        def _():
            scatter_flush(prev, sel)

        @pl.when(change)
        def _():
            for v in range(NV):
                runbuf[nsel * LANES, pl.ds(v * LANES, LANES)] = loadfn(v)

        @pl.when(jnp.logical_not(change))
        def _():
            for v in range(NV):
                runbuf[sel * LANES, pl.ds(v * LANES, LANES)] = (
                    runbuf[sel * LANES, pl.ds(v * LANES, LANES)] + loadfn(v)
                )

        return (rid, nsel)

    def group_step(idrow, b, crow0, go, carry):
        vid = ids_v[idrow, pl.ds(go * LANES, LANES)]
        first = vid[0]
        last = vid[LANES - 1]

        def load(v, j):
            return fbuf[b, crow0 + j, pl.ds(v * LANES, LANES)]

        def uniform_fn(car):
            def groupsum(v):
                s = load(v, 0)
                for j in range(1, LANES):
                    s = s + load(v, j)
                return s

            return absorb(first, groupsum, car)

        def general_fn(car):
            for j in range(LANES):
                car = absorb(vid[j], lambda v, j=j: load(v, j), car)
            return car

        return lax.cond(first == last, uniform_fn, general_fn, carry)

    carry = (jnp.int32(-1), jnp.int32(0))

    GROUPS = CHUNK // LANES

    def chunk_loop(idrow, b, cbase, carry):
        def group_body(go, car):
            return group_step(idrow, b, cbase + go * LANES, go, car)

        return lax.fori_loop(0, GROUPS, group_body, carry)

    def stage_pair(k, carry):
        for b in range(2):
            g = 2 * k + b
            gather(g, b).wait()
            for c in range(CPG):
                carry = chunk_loop(g * CPG + c, b, c * CHUNK, carry)

            @pl.when(g + 2 < NGATHER)
            def _():
                gather(g + 2, b).start()

        return carry

    carry = lax.fori_loop(0, NGATHER // 2, stage_pair, carry)

    pltpu.sync_copy(
        feats_hbm.at[pl.ds(base + NGATHER * STAGE, TAIL), pl.ds(col0, DC)],
        fbuf.at[0, pl.ds(0, TAIL)],
    )
    carry = chunk_loop(NFULL, 0, 0, carry)
    prev, sel = carry
    scatter_flush(prev, sel)
    drain()

    plsc.subcore_barrier()

    pltpu.sync_copy(
        acc.at[pl.ds(sid * 8, 8)],
        out_hbm.at[pl.ds(sid * 8, 8), pl.ds(col0, DC)],
    )


def kernel(feats, segment_ids, num_segments):
    ids = segment_ids.astype(jnp.int32) + (
        jnp.asarray(num_segments, jnp.int32) - G
    )
    main = ids[: (NTILES - 1) * ROWS_PER_TILE].reshape(NTILES - 1, ROWS_PER_TILE)
    last = ids[N - ROWS_PER_TILE :]
    last = jnp.where(
        jnp.arange(ROWS_PER_TILE, dtype=jnp.int32) < OVERLAP, G, last
    )
    ids = jnp.concatenate([main, last[None]], axis=0)
    ids = jnp.pad(
        ids,
        ((0, 0), (0, NCHUNK * CHUNK - ROWS_PER_TILE)),
        constant_values=G,
    )
    ids = ids.reshape(NTILES, NCHUNK, CHUNK)

    mesh = plsc.VectorSubcoreMesh(core_axis_name="c", subcore_axis_name="s")
    run = functools.partial(
        pl.kernel,
        mesh=mesh,
        out_type=jax.ShapeDtypeStruct((G, D), jnp.float32),
        scratch_types=[
            pltpu.VMEM((NCHUNK, CHUNK), jnp.int32),
            pltpu.VMEM((2, STAGE, DC), jnp.float32),
            pltpu.VMEM((8, DC), jnp.float32),
            pltpu.VMEM((2 * LANES, DC), jnp.float32),
            pltpu.VMEM_SHARED((G + 8, DC), jnp.float32),
            pltpu.SemaphoreType.DMA,
            pltpu.SemaphoreType.DMA,
            pltpu.SemaphoreType.DMA,
        ],
    )(_body)
    return run(feats, ids)


# speedup vs baseline: 1.7090x; 1.7090x over previous
import functools

import jax
import jax.numpy as jnp
from jax import lax
from jax.experimental import pallas as pl
from jax.experimental.pallas import tpu as pltpu
from jax.experimental.pallas import tpu_sc as plsc

N = 50000
D = 256
G = 128

NCORES = 2
NTILES = 16
DC = D // NCORES
LANES = 16
NV = DC // LANES
ROWS_PER_TILE = 3128
OVERLAP = NTILES * ROWS_PER_TILE - N
CHUNK = 128
NFULL = ROWS_PER_TILE // CHUNK
TAIL = ROWS_PER_TILE - NFULL * CHUNK
NCHUNK = NFULL + 1
NBUF = 3


def _body(
    feats_hbm, ids_hbm, out_hbm,
    ids_v, fbuf, zbuf, acc, sg0, sg1, sg2, ss0, ss1, ss2,
):
    cid = lax.axis_index("c")
    sid = lax.axis_index("s")
    col0 = cid * DC
    base = jnp.minimum(sid * ROWS_PER_TILE, N - ROWS_PER_TILE)
    sgs = (sg0, sg1, sg2)
    sss = (ss0, ss1, ss2)
    trash_idx = jnp.full((LANES,), G, jnp.int32)

    def gather(j, b, rows=CHUNK):
        return pltpu.make_async_copy(
            feats_hbm.at[pl.ds(base + j * CHUNK, rows), pl.ds(col0, DC)],
            fbuf.at[b] if rows == CHUNK else fbuf.at[b, pl.ds(0, rows)],
            sgs[b],
        )

    def scatter_start(j, b):
        pltpu.async_copy(fbuf.at[b], acc.at[ids_v.at[j]], sss[b], add=True)

    def scatter_wait(b):
        pltpu.make_async_copy(fbuf.at[b], acc.at[ids_v.at[0]], sss[b]).wait()

    for b in range(NBUF):
        gather(b, b).start()

    zero = jnp.zeros((LANES,), jnp.float32)
    for r in range(8):
        for j in range(NV):
            zbuf[r, pl.ds(j * LANES, LANES)] = zero
    pltpu.sync_copy(zbuf, acc.at[pl.ds(sid * 8, 8)])

    pltpu.sync_copy(ids_hbm.at[sid], ids_v)
    plsc.subcore_barrier()

    def triple_body(k, carry):
        for b in range(NBUF):
            j = NBUF * k + b
            gather(j, b).wait()
            scatter_start(j, b)
            pb = (b - 1) % NBUF

            if b == 0:
                @pl.when(k >= 1)
                def _():
                    scatter_wait(pb)
                    gather(j + 2, pb).start()
            else:
                scatter_wait(pb)

                @pl.when(j + 2 < NFULL)
                def _():
                    gather(j + 2, pb).start()

        return carry

    lax.fori_loop(0, NFULL // NBUF, triple_body, 0)

    scatter_wait(NBUF - 1)

    gather(NFULL, 0, TAIL).start()
    gather(NFULL, 0, TAIL).wait()
    scatter_start(NFULL, 0)
    scatter_wait(0)

    plsc.subcore_barrier()

    pltpu.sync_copy(
        acc.at[pl.ds(sid * 8, 8)],
        out_hbm.at[pl.ds(sid * 8, 8), pl.ds(col0, DC)],
    )


def kernel(feats, segment_ids, num_segments):
    ids = segment_ids.astype(jnp.int32) + (
        jnp.asarray(num_segments, jnp.int32) - G
    )
    main = ids[: (NTILES - 1) * ROWS_PER_TILE].reshape(NTILES - 1, ROWS_PER_TILE)
    last = ids[N - ROWS_PER_TILE :]
    last = jnp.where(
        jnp.arange(ROWS_PER_TILE, dtype=jnp.int32) < OVERLAP, G, last
    )
    ids = jnp.concatenate([main, last[None]], axis=0)
    ids = jnp.pad(
        ids,
        ((0, 0), (0, NCHUNK * CHUNK - ROWS_PER_TILE)),
        constant_values=G,
    )
    ids = ids.reshape(NTILES, NCHUNK, CHUNK)

    mesh = plsc.VectorSubcoreMesh(core_axis_name="c", subcore_axis_name="s")
    run = functools.partial(
        pl.kernel,
        mesh=mesh,
        out_type=jax.ShapeDtypeStruct((G, D), jnp.float32),
        scratch_types=[
            pltpu.VMEM((NCHUNK, CHUNK), jnp.int32),
            pltpu.VMEM((NBUF, CHUNK, DC), jnp.float32),
            pltpu.VMEM((8, DC), jnp.float32),
            pltpu.VMEM_SHARED((G + 8, DC), jnp.float32),
            pltpu.SemaphoreType.DMA,
            pltpu.SemaphoreType.DMA,
            pltpu.SemaphoreType.DMA,
            pltpu.SemaphoreType.DMA,
            pltpu.SemaphoreType.DMA,
            pltpu.SemaphoreType.DMA,
        ],
    )(_body)
    return run(feats, ids)


# speedup vs baseline: 2.3613x; 1.3816x over previous
import functools

import jax
import jax.numpy as jnp
from jax import lax
from jax.experimental import pallas as pl
from jax.experimental.pallas import tpu as pltpu
from jax.experimental.pallas import tpu_sc as plsc

N = 50000
D = 256
G = 128

NCORES = 2
NTILES = 16
DC = D // NCORES
LANES = 16
NV = DC // LANES
ROWS_PER_TILE = 3128
OVERLAP = NTILES * ROWS_PER_TILE - N
CHUNK = 128
NFULL = ROWS_PER_TILE // CHUNK
TAIL = ROWS_PER_TILE - NFULL * CHUNK
NCHUNK = NFULL + 1
KPH = 4
AROWS = G * KPH + 16


def _body(
    feats_hbm, ids_hbm, out_hbm,
    ids_v, fbuf, wbuf, zbuf, acc, sg0, sg1,
):
    cid = lax.axis_index("c")
    sid = lax.axis_index("s")
    col0 = cid * DC
    base = jnp.minimum(sid * ROWS_PER_TILE, N - ROWS_PER_TILE)
    sgs = (sg0, sg1)

    def gather(j, b, rows=CHUNK):
        return pltpu.make_async_copy(
            feats_hbm.at[pl.ds(base + j * CHUNK, rows), pl.ds(col0, DC)],
            fbuf.at[b] if rows == CHUNK else fbuf.at[b, pl.ds(0, rows)],
            sgs[b],
        )

    gather(0, 0).start()
    gather(1, 1).start()

    zero = jnp.zeros((LANES,), jnp.float32)
    for r in range(2 * KPH):
        for j in range(NV):
            wbuf[r, pl.ds(j * LANES, LANES)] = zero
    for q in range(4):
        pltpu.sync_copy(
            wbuf.at[pl.ds(0, 8)], acc.at[pl.ds(sid * 8 * KPH + q * 8, 8)]
        )

    @pl.when(sid < 2)
    def _():
        pltpu.sync_copy(
            wbuf.at[pl.ds(0, 8)], acc.at[pl.ds(G * KPH + sid * 8, 8)]
        )

    pltpu.sync_copy(ids_hbm.at[sid], ids_v)
    plsc.subcore_barrier()

    def pair_body(k, carry):
        for b in range(2):
            j = 2 * k + b
            gather(j, b).wait()
            pltpu.sync_copy(fbuf.at[b], acc.at[ids_v.at[j]], add=True)

            @pl.when(j + 2 < NFULL)
            def _():
                gather(j + 2, b).start()

        return carry

    lax.fori_loop(0, NFULL // 2, pair_body, 0)

    gather(NFULL, 0, TAIL).start()
    gather(NFULL, 0, TAIL).wait()
    pltpu.sync_copy(fbuf.at[0], acc.at[ids_v.at[NFULL]], add=True)

    plsc.subcore_barrier()

    pltpu.sync_copy(acc.at[pl.ds(sid * 8 * KPH, 8 * KPH)], wbuf)
    for r in range(8):
        for j in range(NV):
            s = wbuf[r * KPH, pl.ds(j * LANES, LANES)]
            for q in range(1, KPH):
                s = s + wbuf[r * KPH + q, pl.ds(j * LANES, LANES)]
            zbuf[r, pl.ds(j * LANES, LANES)] = s
    pltpu.sync_copy(
        zbuf,
        out_hbm.at[pl.ds(sid * 8, 8), pl.ds(col0, DC)],
    )


def kernel(feats, segment_ids, num_segments):
    ids = segment_ids.astype(jnp.int32) + (
        jnp.asarray(num_segments, jnp.int32) - G
    )
    main = ids[: (NTILES - 1) * ROWS_PER_TILE].reshape(NTILES - 1, ROWS_PER_TILE)
    last = ids[N - ROWS_PER_TILE :]
    last = jnp.where(
        jnp.arange(ROWS_PER_TILE, dtype=jnp.int32) < OVERLAP, G, last
    )
    ids = jnp.concatenate([main, last[None]], axis=0)
    phase = (
        jnp.arange(NTILES * ROWS_PER_TILE, dtype=jnp.int32) % KPH
    ).reshape(NTILES, ROWS_PER_TILE)
    ids = jnp.where(ids >= G, G * KPH, ids * KPH + phase)
    ids = jnp.pad(
        ids,
        ((0, 0), (0, NCHUNK * CHUNK - ROWS_PER_TILE)),
        constant_values=G * KPH,
    )
    ids = ids.reshape(NTILES, NCHUNK, CHUNK)

    mesh = plsc.VectorSubcoreMesh(core_axis_name="c", subcore_axis_name="s")
    run = functools.partial(
        pl.kernel,
        mesh=mesh,
        out_type=jax.ShapeDtypeStruct((G, D), jnp.float32),
        scratch_types=[
            pltpu.VMEM((NCHUNK, CHUNK), jnp.int32),
            pltpu.VMEM((2, CHUNK, DC), jnp.float32),
            pltpu.VMEM((8 * KPH, DC), jnp.float32),
            pltpu.VMEM((8, DC), jnp.float32),
            pltpu.VMEM_SHARED((AROWS, DC), jnp.float32),
            pltpu.SemaphoreType.DMA,
            pltpu.SemaphoreType.DMA,
        ],
    )(_body)
    return run(feats, ids)
